# Initial kernel scaffold; baseline (speedup 1.0000x reference)
import functools

import jax
import jax.numpy as jnp
from jax import lax
from jax.experimental import pallas as pl
from jax.experimental.pallas import tpu as pltpu
from jax.experimental.pallas import tpu_sc as plsc

G = 256
NCORE = 2
NSUB = 16
NW = NCORE * NSUB
CHUNK = 80


def _edge_embed_body(ea_ref, we_ref, e_ref):
    e_ref[...] = jax.nn.relu(
        jnp.dot(ea_ref[...], we_ref[...], preferred_element_type=jnp.float32))


def _edge_embed(edge_attr, W_edge, block_e):
    E, DE = edge_attr.shape
    D = W_edge.shape[1]
    return pl.pallas_call(
        _edge_embed_body,
        grid=(E // block_e,),
        in_specs=[
            pl.BlockSpec((block_e, DE), lambda i: (i, 0)),
            pl.BlockSpec((DE, D), lambda i: (0, 0)),
        ],
        out_specs=pl.BlockSpec((block_e, D), lambda i: (i, 0)),
        out_shape=jax.ShapeDtypeStruct((E, D), jnp.float32),
    )(edge_attr, W_edge)


def _mlp_body(hin_ref, p0_ref, p1_ref, e0_ref, e1_ref, b_ref,
              w1_ref, b1_ref, w2_ref, b2_ref, hout_ref, g_ref, *, R):
    i = pl.program_id(0)
    hin = hin_ref[...]
    u = hin + p0_ref[...] + p1_ref[...] + e0_ref[...] + e1_ref[...]
    t = jax.nn.relu(
        jnp.dot(u, w1_ref[...], preferred_element_type=jnp.float32) + b1_ref[...])
    z = jnp.dot(t, w2_ref[...], preferred_element_type=jnp.float32) + b2_ref[...]
    h = hin + jax.nn.relu(z)
    hout_ref[...] = h
    P = (lax.broadcasted_iota(jnp.int32, (R, G), 1) == b_ref[...]).astype(jnp.float32)
    gg = lax.dot_general(P, h, (((0,), (0,)), ((), ())),
                         preferred_element_type=jnp.float32)

    @pl.when(i == 0)
    def _():
        g_ref[...] = gg

    @pl.when(i != 0)
    def _():
        g_ref[...] += gg


def _mlp(hin, p0, p1, e0, e1, batch2d, W1l, b1l, W2l, b2l, R):
    N, D = hin.shape
    D2 = W1l.shape[1]
    return pl.pallas_call(
        functools.partial(_mlp_body, R=R),
        grid=(N // R,),
        in_specs=[
            pl.BlockSpec((R, D), lambda i: (i, 0)),
            pl.BlockSpec((R, D), lambda i: (i, 0)),
            pl.BlockSpec((R, D), lambda i: (i, 0)),
            pl.BlockSpec((R, D), lambda i: (i, 0)),
            pl.BlockSpec((R, D), lambda i: (i, 0)),
            pl.BlockSpec((R, 1), lambda i: (i, 0)),
            pl.BlockSpec((D, D2), lambda i: (0, 0)),
            pl.BlockSpec((1, D2), lambda i: (0, 0)),
            pl.BlockSpec((D2, D), lambda i: (0, 0)),
            pl.BlockSpec((1, D), lambda i: (0, 0)),
        ],
        out_specs=[
            pl.BlockSpec((R, D), lambda i: (i, 0)),
            pl.BlockSpec((G, D), lambda i: (0, 0)),
        ],
        out_shape=[
            jax.ShapeDtypeStruct((N, D), jnp.float32),
            jax.ShapeDtypeStruct((G, D), jnp.float32),
        ],
    )(hin, p0, p1, e0, e1, batch2d, W1l, b1l, W2l, b2l)


def _vn_body(g_ref, vn_ref, wv_ref, bv_ref, out_ref):
    out_ref[...] = vn_ref[...] + jax.nn.relu(
        jnp.dot(g_ref[...], wv_ref[...], preferred_element_type=jnp.float32)
        + bv_ref[...])


def _vn_update(g, vn, Wvl, bvl):
    return pl.pallas_call(
        _vn_body,
        out_shape=jax.ShapeDtypeStruct(vn.shape, jnp.float32),
    )(g, vn, Wvl, bvl)


def _hin_body(h_ref, vn_ref, b_ref, out_ref, *, R):
    P = (lax.broadcasted_iota(jnp.int32, (R, G), 1) == b_ref[...]).astype(jnp.float32)
    out_ref[...] = h_ref[...] + jnp.dot(P, vn_ref[...],
                                        preferred_element_type=jnp.float32)


def _hin(h, vn, batch2d, R):
    N, D = h.shape
    return pl.pallas_call(
        functools.partial(_hin_body, R=R),
        grid=(N // R,),
        in_specs=[
            pl.BlockSpec((R, D), lambda i: (i, 0)),
            pl.BlockSpec((G, D), lambda i: (0, 0)),
            pl.BlockSpec((R, 1), lambda i: (i, 0)),
        ],
        out_specs=pl.BlockSpec((R, D), lambda i: (i, 0)),
        out_shape=jax.ShapeDtypeStruct((N, D), jnp.float32),
    )(h, vn, batch2d)


def _pool_combine_body(pp_ref, out_ref):
    m = jnp.max(pp_ref[...], axis=0)
    out_ref[...] = jnp.where(jnp.isfinite(m), m, 0.0)


def _pool_combine(pp):
    _, g, d = pp.shape
    return pl.pallas_call(
        _pool_combine_body,
        out_shape=jax.ShapeDtypeStruct((g, d), jnp.float32),
    )(pp)


def _sc_segment_sum(rows_hbm, dstT, N, srcT=None):
    D = rows_hbm.shape[1]
    NCH, C = dstT.shape[1], dstT.shape[2]
    gather = srcT is not None
    rpt = N // NSUB
    ZR = 125
    assert rpt % ZR == 0
    mesh = plsc.VectorSubcoreMesh(core_axis_name="c", subcore_axis_name="s")

    scratch = [
        pltpu.VMEM((NCH, C), jnp.int32),
        pltpu.VMEM((C, D), jnp.float32),
        pltpu.VMEM((ZR, D), jnp.float32),
        pltpu.VMEM_SHARED((N, D), jnp.float32),
    ]
    if gather:
        scratch.insert(1, pltpu.VMEM((NCH, C), jnp.int32))

    @functools.partial(
        pl.kernel,
        out_type=jax.ShapeDtypeStruct((NCORE, N, D), jnp.float32),
        mesh=mesh,
        scratch_types=scratch,
    )
    def k(*refs):
        if gather:
            rows_ref, src_hbm, dst_hbm, out_hbm, dst_v, src_v, rows_v, zbuf, acc = refs
        else:
            rows_ref, dst_hbm, out_hbm, dst_v, rows_v, zbuf, acc = refs
        c = lax.axis_index("c")
        s = lax.axis_index("s")
        wid = c * NSUB + s

        @pl.loop(0, ZR)
        def _(i):
            @pl.loop(0, D // 16)
            def _(j):
                zbuf[i, pl.ds(j * 16, 16)] = jnp.zeros((16,), jnp.float32)

        @pl.loop(0, rpt // ZR)
        def _(i):
            pltpu.sync_copy(zbuf, acc.at[pl.ds(s * rpt + i * ZR, ZR)])

        plsc.subcore_barrier()

        pltpu.sync_copy(dst_hbm.at[wid], dst_v)
        if gather:
            pltpu.sync_copy(src_hbm.at[wid], src_v)

        @pl.loop(0, NCH)
        def _(j):
            if gather:
                pltpu.sync_copy(rows_ref.at[src_v.at[j]], rows_v)
            else:
                pltpu.sync_copy(rows_ref.at[pl.ds((wid * NCH + j) * C, C)], rows_v)
            pltpu.sync_copy(rows_v, acc.at[dst_v.at[j]], add=True)

        plsc.subcore_barrier()
        pltpu.sync_copy(acc.at[pl.ds(s * rpt, rpt)],
                        out_hbm.at[c, pl.ds(s * rpt, rpt)])

    if gather:
        return k(rows_hbm, srcT, dstT)
    return k(rows_hbm, dstT)


def _sc_segment_max(h, batchT):
    N, D = h.shape
    NCHUNKS, CH = batchT.shape
    STEPS = -(-NCHUNKS // NW)
    mesh = plsc.VectorSubcoreMesh(core_axis_name="c", subcore_axis_name="s")

    @functools.partial(
        pl.kernel,
        out_type=jax.ShapeDtypeStruct((NW, G, D), jnp.float32),
        mesh=mesh,
        scratch_types=[
            pltpu.VMEM((NCHUNKS, CH), jnp.int32),
            pltpu.VMEM((CH, D), jnp.float32),
            pltpu.VMEM((G, D), jnp.float32),
        ],
    )
    def k(h_hbm, b_hbm, out_hbm, b_v, rows_v, acc):
        c = lax.axis_index("c")
        s = lax.axis_index("s")
        wid = c * NSUB + s

        @pl.loop(0, G)
        def _(i):
            @pl.loop(0, D // 16)
            def _(j):
                acc[i, pl.ds(j * 16, 16)] = jnp.full((16,), -jnp.inf, jnp.float32)

        pltpu.sync_copy(b_hbm, b_v)
        lane = lax.iota(jnp.int32, 16)

        @pl.loop(0, STEPS)
        def _(t):
            chunk = t * NW + wid

            @pl.when(chunk < NCHUNKS)
            def _():
                pltpu.sync_copy(h_hbm.at[pl.ds(chunk * CH, CH)], rows_v)
                bvec = b_v[chunk, pl.ds(0, CH)]
                for i in range(CH):
                    bi = jnp.max(jnp.where(lane == i, bvec, -1))
                    for j in range(D // 16):
                        sl = pl.ds(j * 16, 16)
                        acc[bi, sl] = jnp.maximum(acc[bi, sl], rows_v[i, sl])

        pltpu.sync_copy(acc, out_hbm.at[wid])

    return k(h, batchT)


def kernel(x, edge_attr, W_edge, W1, b1, W2, b2, Wv, bv, edge_index, batch):
    N, D = x.shape
    E = edge_attr.shape[0]
    L = W1.shape[0]
    R = 1000
    assert N % R == 0 and N % NSUB == 0 and N % 16 == 0
    NCH = E // (NW * CHUNK)
    assert NW * NCH * CHUNK == E

    src = edge_index[0].astype(jnp.int32)
    dst = edge_index[1].astype(jnp.int32)
    srcT = src.reshape(NW, NCH, CHUNK)
    dstT = dst.reshape(NW, NCH, CHUNK)
    b32 = batch.astype(jnp.int32)
    batch2d = b32.reshape(N, 1)
    batchT = b32.reshape(N // 16, 16)

    e = _edge_embed(edge_attr, W_edge, block_e=2000)
    es = _sc_segment_sum(e, dstT, N)

    h = x
    vn = jnp.zeros((G, D), dtype=jnp.float32)
    for l in range(L):
        hin = h if l == 0 else _hin(h, vn, batch2d, R)
        p = _sc_segment_sum(hin, dstT, N, srcT=srcT)
        h, g = _mlp(hin, p[0], p[1], es[0], es[1], batch2d,
                    W1[l], b1[l][None], W2[l], b2[l][None], R)
        vn = _vn_update(g, vn, Wv[l], bv[l][None])

    pp = _sc_segment_max(h, batchT)
    return _pool_combine(pp)


# speedup vs baseline: 4.9613x; 4.9613x over previous
import dataclasses
import functools

import jax
import jax.numpy as jnp
from jax import lax
from jax.experimental import pallas as pl
from jax.experimental.pallas import tpu as pltpu
from jax.experimental.pallas import tpu_sc as plsc

G = 256
NCORE = 2
NSUB = 16
NW = NCORE * NSUB
CHUNK = 80


def _sc_compiler_params():
    cp = pltpu.CompilerParams()
    if "needs_layout_passes" in pltpu.CompilerParams.__dataclass_fields__:
        cp = dataclasses.replace(cp, needs_layout_passes=False)
    return cp


def _edge_embed_body(ea_ref, we_ref, e_ref):
    e_ref[...] = jax.nn.relu(
        jnp.dot(ea_ref[...], we_ref[...], preferred_element_type=jnp.float32))


def _edge_embed(edge_attr, W_edge, block_e):
    E, DE = edge_attr.shape
    D = W_edge.shape[1]
    return pl.pallas_call(
        _edge_embed_body,
        grid=(E // block_e,),
        in_specs=[
            pl.BlockSpec((block_e, DE), lambda i: (i, 0)),
            pl.BlockSpec((DE, D), lambda i: (0, 0)),
        ],
        out_specs=pl.BlockSpec((block_e, D), lambda i: (i, 0)),
        out_shape=jax.ShapeDtypeStruct((E, D), jnp.float32),
    )(edge_attr, W_edge)


def _mlp_body(hin_ref, p0_ref, p1_ref, e0_ref, e1_ref, b_ref,
              w1_ref, b1_ref, w2_ref, b2_ref, hout_ref, g_ref, *, R):
    i = pl.program_id(0)
    hin = hin_ref[...]
    u = hin + p0_ref[...] + p1_ref[...] + e0_ref[...] + e1_ref[...]
    t = jax.nn.relu(
        jnp.dot(u, w1_ref[...], preferred_element_type=jnp.float32) + b1_ref[...])
    z = jnp.dot(t, w2_ref[...], preferred_element_type=jnp.float32) + b2_ref[...]
    h = hin + jax.nn.relu(z)
    hout_ref[...] = h
    P = (lax.broadcasted_iota(jnp.int32, (R, G), 1) == b_ref[...]).astype(jnp.float32)
    gg = lax.dot_general(P, h, (((0,), (0,)), ((), ())),
                         preferred_element_type=jnp.float32)

    @pl.when(i == 0)
    def _():
        g_ref[...] = gg

    @pl.when(i != 0)
    def _():
        g_ref[...] += gg


def _mlp(hin, p0, p1, e0, e1, batch2d, W1l, b1l, W2l, b2l, R):
    N, D = hin.shape
    D2 = W1l.shape[1]
    return pl.pallas_call(
        functools.partial(_mlp_body, R=R),
        grid=(N // R,),
        in_specs=[
            pl.BlockSpec((R, D), lambda i: (i, 0)),
            pl.BlockSpec((R, D), lambda i: (i, 0)),
            pl.BlockSpec((R, D), lambda i: (i, 0)),
            pl.BlockSpec((R, D), lambda i: (i, 0)),
            pl.BlockSpec((R, D), lambda i: (i, 0)),
            pl.BlockSpec((R, 1), lambda i: (i, 0)),
            pl.BlockSpec((D, D2), lambda i: (0, 0)),
            pl.BlockSpec((1, D2), lambda i: (0, 0)),
            pl.BlockSpec((D2, D), lambda i: (0, 0)),
            pl.BlockSpec((1, D), lambda i: (0, 0)),
        ],
        out_specs=[
            pl.BlockSpec((R, D), lambda i: (i, 0)),
            pl.BlockSpec((G, D), lambda i: (0, 0)),
        ],
        out_shape=[
            jax.ShapeDtypeStruct((N, D), jnp.float32),
            jax.ShapeDtypeStruct((G, D), jnp.float32),
        ],
    )(hin, p0, p1, e0, e1, batch2d, W1l, b1l, W2l, b2l)


def _vn_body(g_ref, vn_ref, wv_ref, bv_ref, out_ref):
    out_ref[...] = vn_ref[...] + jax.nn.relu(
        jnp.dot(g_ref[...], wv_ref[...], preferred_element_type=jnp.float32)
        + bv_ref[...])


def _vn_update(g, vn, Wvl, bvl):
    return pl.pallas_call(
        _vn_body,
        out_shape=jax.ShapeDtypeStruct(vn.shape, jnp.float32),
    )(g, vn, Wvl, bvl)


def _hin_body(h_ref, vn_ref, b_ref, out_ref, *, R):
    P = (lax.broadcasted_iota(jnp.int32, (R, G), 1) == b_ref[...]).astype(jnp.float32)
    out_ref[...] = h_ref[...] + jnp.dot(P, vn_ref[...],
                                        preferred_element_type=jnp.float32)


def _hin(h, vn, batch2d, R):
    N, D = h.shape
    return pl.pallas_call(
        functools.partial(_hin_body, R=R),
        grid=(N // R,),
        in_specs=[
            pl.BlockSpec((R, D), lambda i: (i, 0)),
            pl.BlockSpec((G, D), lambda i: (0, 0)),
            pl.BlockSpec((R, 1), lambda i: (i, 0)),
        ],
        out_specs=pl.BlockSpec((R, D), lambda i: (i, 0)),
        out_shape=jax.ShapeDtypeStruct((N, D), jnp.float32),
    )(h, vn, batch2d)


def _pool_combine_body(pp_ref, out_ref):
    m = jnp.max(pp_ref[...], axis=0)
    out_ref[...] = jnp.where(jnp.isfinite(m), m, 0.0)


def _pool_combine(pp):
    _, g, d = pp.shape
    return pl.pallas_call(
        _pool_combine_body,
        out_shape=jax.ShapeDtypeStruct((g, d), jnp.float32),
    )(pp)


def _sc_segment_sum(rows_hbm, dstT, N, srcT=None):
    D = rows_hbm.shape[1]
    NCH, C = dstT.shape[1], dstT.shape[2]
    gather = srcT is not None
    rpt = (N // NSUB) // 8 * 8
    tail = N - NSUB * rpt
    ZC, ZT = divmod(rpt, C)
    assert ZT % 8 == 0 and tail % 8 == 0 and tail <= C
    mesh = plsc.VectorSubcoreMesh(core_axis_name="c", subcore_axis_name="s")

    scratch = [
        pltpu.VMEM((NCH, C), jnp.int32),
        pltpu.VMEM((C, D), jnp.float32),
        pltpu.VMEM_SHARED((N, D), jnp.float32),
    ]
    if gather:
        scratch.insert(1, pltpu.VMEM((NCH, C), jnp.int32))

    @functools.partial(
        pl.kernel,
        out_type=jax.ShapeDtypeStruct((NCORE, N, D), jnp.float32),
        mesh=mesh,
        scratch_types=scratch,
        compiler_params=_sc_compiler_params(),
    )
    def k(*refs):
        if gather:
            rows_ref, src_hbm, dst_hbm, out_hbm, dst_v, src_v, rows_v, acc = refs
        else:
            rows_ref, dst_hbm, out_hbm, dst_v, rows_v, acc = refs
        c = lax.axis_index("c")
        s = lax.axis_index("s")
        wid = c * NSUB + s

        @pl.loop(0, C)
        def _(i):
            @pl.loop(0, D // 16)
            def _(j):
                rows_v[i, pl.ds(j * 16, 16)] = jnp.zeros((16,), jnp.float32)

        @pl.loop(0, ZC)
        def _(i):
            pltpu.sync_copy(rows_v, acc.at[pl.ds(s * rpt + i * C, C)])

        if ZT:
            pltpu.sync_copy(rows_v.at[pl.ds(0, ZT)],
                            acc.at[pl.ds(s * rpt + ZC * C, ZT)])

        if tail:
            @pl.when(s == NSUB - 1)
            def _():
                pltpu.sync_copy(rows_v.at[pl.ds(0, tail)],
                                acc.at[pl.ds(NSUB * rpt, tail)])

        plsc.subcore_barrier()

        pltpu.sync_copy(dst_hbm.at[wid], dst_v)
        if gather:
            pltpu.sync_copy(src_hbm.at[wid], src_v)

        @pl.loop(0, NCH)
        def _(j):
            if gather:
                pltpu.sync_copy(rows_ref.at[src_v.at[j]], rows_v)
            else:
                pltpu.sync_copy(rows_ref.at[pl.ds((wid * NCH + j) * C, C)], rows_v)
            pltpu.sync_copy(rows_v, acc.at[dst_v.at[j]], add=True)

        plsc.subcore_barrier()
        pltpu.sync_copy(acc.at[pl.ds(s * rpt, rpt)],
                        out_hbm.at[c, pl.ds(s * rpt, rpt)])
        if tail:
            @pl.when(s == NSUB - 1)
            def _():
                pltpu.sync_copy(acc.at[pl.ds(NSUB * rpt, tail)],
                                out_hbm.at[c, pl.ds(NSUB * rpt, tail)])

    if gather:
        return k(rows_hbm, srcT, dstT)
    return k(rows_hbm, dstT)


def _sc_segment_max(h, batchT):
    N, D = h.shape
    NCHUNKS, CH = batchT.shape
    STEPS = -(-NCHUNKS // NW)
    mesh = plsc.VectorSubcoreMesh(core_axis_name="c", subcore_axis_name="s")

    @functools.partial(
        pl.kernel,
        out_type=jax.ShapeDtypeStruct((NW, G, D), jnp.float32),
        mesh=mesh,
        scratch_types=[
            pltpu.VMEM((NCHUNKS, CH), jnp.int32),
            pltpu.VMEM((CH, D), jnp.float32),
            pltpu.VMEM((G, D), jnp.float32),
        ],
        compiler_params=_sc_compiler_params(),
    )
    def k(h_hbm, b_hbm, out_hbm, b_v, rows_v, acc):
        c = lax.axis_index("c")
        s = lax.axis_index("s")
        wid = c * NSUB + s

        @pl.loop(0, G)
        def _(i):
            @pl.loop(0, D // 16)
            def _(j):
                acc[i, pl.ds(j * 16, 16)] = jnp.full((16,), -jnp.inf, jnp.float32)

        pltpu.sync_copy(b_hbm, b_v)
        lane = lax.iota(jnp.int32, 16)

        @pl.loop(0, STEPS)
        def _(t):
            chunk = t * NW + wid

            @pl.when(chunk < NCHUNKS)
            def _():
                pltpu.sync_copy(h_hbm.at[pl.ds(chunk * CH, CH)], rows_v)
                bvec = b_v[chunk, pl.ds(0, CH)]
                for i in range(CH):
                    bi = jnp.max(jnp.where(lane == i, bvec, -1))
                    for j in range(D // 16):
                        sl = pl.ds(j * 16, 16)
                        acc[bi, sl] = jnp.maximum(acc[bi, sl], rows_v[i, sl])

        pltpu.sync_copy(acc, out_hbm.at[wid])

    return k(h, batchT)


def kernel(x, edge_attr, W_edge, W1, b1, W2, b2, Wv, bv, edge_index, batch):
    N, D = x.shape
    E = edge_attr.shape[0]
    L = W1.shape[0]
    R = 1000
    assert N % R == 0 and N % NSUB == 0 and N % 16 == 0
    NCH = E // (NW * CHUNK)
    assert NW * NCH * CHUNK == E

    src = edge_index[0].astype(jnp.int32)
    dst = edge_index[1].astype(jnp.int32)
    srcT = src.reshape(NW, NCH, CHUNK)
    dstT = dst.reshape(NW, NCH, CHUNK)
    b32 = batch.astype(jnp.int32)
    batch2d = b32.reshape(N, 1)
    batchT = b32.reshape(N // 16, 16)

    e = _edge_embed(edge_attr, W_edge, block_e=2000)
    es = _sc_segment_sum(e, dstT, N)

    h = x
    vn = jnp.zeros((G, D), dtype=jnp.float32)
    for l in range(L):
        hin = h if l == 0 else _hin(h, vn, batch2d, R)
        p = _sc_segment_sum(hin, dstT, N, srcT=srcT)
        h, g = _mlp(hin, p[0], p[1], es[0], es[1], batch2d,
                    W1[l], b1[l][None], W2[l], b2[l][None], R)
        vn = _vn_update(g, vn, Wv[l], bv[l][None])

    pp = _sc_segment_max(h, batchT)
    return _pool_combine(pp)


# speedup vs baseline: 6.8119x; 1.3730x over previous
import dataclasses
import functools

import jax
import jax.numpy as jnp
from jax import lax
from jax.experimental import pallas as pl
from jax.experimental.pallas import tpu as pltpu
from jax.experimental.pallas import tpu_sc as plsc

G = 256
NCORE = 2
NSUB = 16
NW = NCORE * NSUB
CHUNK = 80


def _sc_compiler_params():
    cp = pltpu.CompilerParams()
    if "needs_layout_passes" in pltpu.CompilerParams.__dataclass_fields__:
        cp = dataclasses.replace(cp, needs_layout_passes=False)
    return cp


def _edge_embed_body(ea_ref, we_ref, e_ref):
    e_ref[...] = jax.nn.relu(
        jnp.dot(ea_ref[...], we_ref[...], preferred_element_type=jnp.float32))


def _edge_embed(edge_attr, W_edge, block_e):
    E, DE = edge_attr.shape
    D = W_edge.shape[1]
    return pl.pallas_call(
        _edge_embed_body,
        grid=(E // block_e,),
        in_specs=[
            pl.BlockSpec((block_e, DE), lambda i: (i, 0)),
            pl.BlockSpec((DE, D), lambda i: (0, 0)),
        ],
        out_specs=pl.BlockSpec((block_e, D), lambda i: (i, 0)),
        out_shape=jax.ShapeDtypeStruct((E, D), jnp.float32),
    )(edge_attr, W_edge)


def _mlp_body(hin_ref, p0_ref, p1_ref, e0_ref, e1_ref, b_ref,
              w1_ref, b1_ref, w2_ref, b2_ref, hout_ref, g_ref, *, R):
    i = pl.program_id(0)
    hin = hin_ref[...]
    u = hin + p0_ref[...] + p1_ref[...] + e0_ref[...] + e1_ref[...]
    t = jax.nn.relu(
        jnp.dot(u, w1_ref[...], preferred_element_type=jnp.float32) + b1_ref[...])
    z = jnp.dot(t, w2_ref[...], preferred_element_type=jnp.float32) + b2_ref[...]
    h = hin + jax.nn.relu(z)
    hout_ref[...] = h
    P = (lax.broadcasted_iota(jnp.int32, (R, G), 1) == b_ref[...]).astype(jnp.float32)
    gg = lax.dot_general(P, h, (((0,), (0,)), ((), ())),
                         preferred_element_type=jnp.float32)

    @pl.when(i == 0)
    def _():
        g_ref[...] = gg

    @pl.when(i != 0)
    def _():
        g_ref[...] += gg


def _mlp(hin, p0, p1, e0, e1, batch2d, W1l, b1l, W2l, b2l, R):
    N, D = hin.shape
    D2 = W1l.shape[1]
    return pl.pallas_call(
        functools.partial(_mlp_body, R=R),
        grid=(N // R,),
        in_specs=[
            pl.BlockSpec((R, D), lambda i: (i, 0)),
            pl.BlockSpec((R, D), lambda i: (i, 0)),
            pl.BlockSpec((R, D), lambda i: (i, 0)),
            pl.BlockSpec((R, D), lambda i: (i, 0)),
            pl.BlockSpec((R, D), lambda i: (i, 0)),
            pl.BlockSpec((R, 1), lambda i: (i, 0)),
            pl.BlockSpec((D, D2), lambda i: (0, 0)),
            pl.BlockSpec((1, D2), lambda i: (0, 0)),
            pl.BlockSpec((D2, D), lambda i: (0, 0)),
            pl.BlockSpec((1, D), lambda i: (0, 0)),
        ],
        out_specs=[
            pl.BlockSpec((R, D), lambda i: (i, 0)),
            pl.BlockSpec((G, D), lambda i: (0, 0)),
        ],
        out_shape=[
            jax.ShapeDtypeStruct((N, D), jnp.float32),
            jax.ShapeDtypeStruct((G, D), jnp.float32),
        ],
    )(hin, p0, p1, e0, e1, batch2d, W1l, b1l, W2l, b2l)


def _vn_body(g_ref, vn_ref, wv_ref, bv_ref, out_ref):
    out_ref[...] = vn_ref[...] + jax.nn.relu(
        jnp.dot(g_ref[...], wv_ref[...], preferred_element_type=jnp.float32)
        + bv_ref[...])


def _vn_update(g, vn, Wvl, bvl):
    return pl.pallas_call(
        _vn_body,
        out_shape=jax.ShapeDtypeStruct(vn.shape, jnp.float32),
    )(g, vn, Wvl, bvl)


def _hin_body(h_ref, vn_ref, b_ref, out_ref, *, R):
    P = (lax.broadcasted_iota(jnp.int32, (R, G), 1) == b_ref[...]).astype(jnp.float32)
    out_ref[...] = h_ref[...] + jnp.dot(P, vn_ref[...],
                                        preferred_element_type=jnp.float32)


def _hin(h, vn, batch2d, R):
    N, D = h.shape
    return pl.pallas_call(
        functools.partial(_hin_body, R=R),
        grid=(N // R,),
        in_specs=[
            pl.BlockSpec((R, D), lambda i: (i, 0)),
            pl.BlockSpec((G, D), lambda i: (0, 0)),
            pl.BlockSpec((R, 1), lambda i: (i, 0)),
        ],
        out_specs=pl.BlockSpec((R, D), lambda i: (i, 0)),
        out_shape=jax.ShapeDtypeStruct((N, D), jnp.float32),
    )(h, vn, batch2d)


def _pool_combine_body(pp_ref, out_ref):
    m = jnp.max(pp_ref[...], axis=0)
    out_ref[...] = jnp.where(jnp.isfinite(m), m, 0.0)


def _pool_combine(pp):
    _, g, d = pp.shape
    return pl.pallas_call(
        _pool_combine_body,
        out_shape=jax.ShapeDtypeStruct((g, d), jnp.float32),
    )(pp)


def _sc_segment_sum(rows_hbm, dstT, N, srcT=None):
    D = rows_hbm.shape[1]
    NSC, SB, C = dstT.shape[1], dstT.shape[2], dstT.shape[3]
    NCH = NSC * SB
    gather = srcT is not None
    rpt = (N // NSUB) // 8 * 8
    tail = N - NSUB * rpt
    ZC, ZT = divmod(rpt, C)
    assert ZT % 8 == 0 and tail % 8 == 0 and tail <= C
    mesh = plsc.VectorSubcoreMesh(core_axis_name="c", subcore_axis_name="s")

    scratch = [
        pltpu.VMEM((SB, C), jnp.int32),
        pltpu.VMEM((C, D), jnp.float32),
        pltpu.VMEM((C, D), jnp.float32),
        pltpu.VMEM_SHARED((N, D), jnp.float32),
        pltpu.SemaphoreType.DMA,
        pltpu.SemaphoreType.DMA,
    ]
    if gather:
        scratch.insert(1, pltpu.VMEM((SB, C), jnp.int32))

    @functools.partial(
        pl.kernel,
        out_type=jax.ShapeDtypeStruct((NCORE, N, D), jnp.float32),
        mesh=mesh,
        scratch_types=scratch,
        compiler_params=_sc_compiler_params(),
    )
    def k(*refs):
        if gather:
            rows_ref, src_hbm, dst_hbm, out_hbm, dst_v, src_v, r0, r1, acc, g0, g1 = refs
        else:
            rows_ref, dst_hbm, out_hbm, dst_v, r0, r1, acc, g0, g1 = refs
        c = lax.axis_index("c")
        s = lax.axis_index("s")
        wid = c * NSUB + s

        @pl.loop(0, C)
        def _(i):
            @pl.loop(0, D // 16)
            def _(j):
                r0[i, pl.ds(j * 16, 16)] = jnp.zeros((16,), jnp.float32)

        @pl.loop(0, ZC)
        def _(i):
            pltpu.sync_copy(r0, acc.at[pl.ds(s * rpt + i * C, C)])

        if ZT:
            pltpu.sync_copy(r0.at[pl.ds(0, ZT)],
                            acc.at[pl.ds(s * rpt + ZC * C, ZT)])

        if tail:
            @pl.when(s == NSUB - 1)
            def _():
                pltpu.sync_copy(r0.at[pl.ds(0, tail)],
                                acc.at[pl.ds(NSUB * rpt, tail)])

        plsc.subcore_barrier()

        assert SB % 2 == 1 and SB >= 3

        @pl.loop(0, NSC)
        def _(u):
            pltpu.sync_copy(dst_hbm.at[wid, u], dst_v)
            if gather:
                pltpu.sync_copy(src_hbm.at[wid, u], src_v)

            def load_copy(j, buf, sem):
                if gather:
                    return pltpu.make_async_copy(
                        rows_ref.at[src_v.at[j]], buf, sem)
                return pltpu.make_async_copy(
                    rows_ref.at[pl.ds(((wid * NSC + u) * SB + j) * C, C)],
                    buf, sem)

            def add_from(j, buf):
                pltpu.sync_copy(buf, acc.at[dst_v.at[j]], add=True)

            load_copy(0, r0, g0).start()
            load_copy(1, r1, g1).start()

            @pl.loop(0, SB - 1, step=2)
            def _(j):
                load_copy(j, r0, g0).wait()
                add_from(j, r0)
                load_copy(j + 2, r0, g0).start()
                load_copy(j + 1, r1, g1).wait()
                add_from(j + 1, r1)

                @pl.when(j + 3 < SB)
                def _():
                    load_copy(j + 3, r1, g1).start()

            load_copy(SB - 1, r0, g0).wait()
            add_from(SB - 1, r0)

        plsc.subcore_barrier()
        pltpu.sync_copy(acc.at[pl.ds(s * rpt, rpt)],
                        out_hbm.at[c, pl.ds(s * rpt, rpt)])
        if tail:
            @pl.when(s == NSUB - 1)
            def _():
                pltpu.sync_copy(acc.at[pl.ds(NSUB * rpt, tail)],
                                out_hbm.at[c, pl.ds(NSUB * rpt, tail)])

    if gather:
        return k(rows_hbm, srcT, dstT)
    return k(rows_hbm, dstT)


def _sc_segment_max(h, batchT):
    N, D = h.shape
    NCHUNKS, CH = batchT.shape
    STEPS = -(-NCHUNKS // NW)
    mesh = plsc.VectorSubcoreMesh(core_axis_name="c", subcore_axis_name="s")

    @functools.partial(
        pl.kernel,
        out_type=jax.ShapeDtypeStruct((NW, G, D), jnp.float32),
        mesh=mesh,
        scratch_types=[
            pltpu.VMEM((NCHUNKS, CH), jnp.int32),
            pltpu.VMEM((CH, D), jnp.float32),
            pltpu.VMEM((G, D), jnp.float32),
        ],
        compiler_params=_sc_compiler_params(),
    )
    def k(h_hbm, b_hbm, out_hbm, b_v, rows_v, acc):
        c = lax.axis_index("c")
        s = lax.axis_index("s")
        wid = c * NSUB + s

        @pl.loop(0, G)
        def _(i):
            @pl.loop(0, D // 16)
            def _(j):
                acc[i, pl.ds(j * 16, 16)] = jnp.full((16,), -jnp.inf, jnp.float32)

        pltpu.sync_copy(b_hbm, b_v)
        lane = lax.iota(jnp.int32, 16)

        @pl.loop(0, STEPS)
        def _(t):
            chunk = t * NW + wid

            @pl.when(chunk < NCHUNKS)
            def _():
                pltpu.sync_copy(h_hbm.at[pl.ds(chunk * CH, CH)], rows_v)
                bvec = b_v[chunk, pl.ds(0, CH)]
                for i in range(CH):
                    bi = jnp.max(jnp.where(lane == i, bvec, -1))
                    for j in range(D // 16):
                        sl = pl.ds(j * 16, 16)
                        acc[bi, sl] = jnp.maximum(acc[bi, sl], rows_v[i, sl])

        pltpu.sync_copy(acc, out_hbm.at[wid])

    return k(h, batchT)


def kernel(x, edge_attr, W_edge, W1, b1, W2, b2, Wv, bv, edge_index, batch):
    N, D = x.shape
    E = edge_attr.shape[0]
    L = W1.shape[0]
    R = 1000
    assert N % R == 0 and N % NSUB == 0 and N % 16 == 0
    NCH = E // (NW * CHUNK)
    assert NW * NCH * CHUNK == E
    SB = 25
    NSC = NCH // SB
    assert NSC * SB == NCH

    src = edge_index[0].astype(jnp.int32)
    dst = edge_index[1].astype(jnp.int32)
    srcT = src.reshape(NW, NSC, SB, CHUNK)
    dstT = dst.reshape(NW, NSC, SB, CHUNK)
    b32 = batch.astype(jnp.int32)
    batch2d = b32.reshape(N, 1)
    batchT = b32.reshape(N // 16, 16)

    e = _edge_embed(edge_attr, W_edge, block_e=2000)
    es = _sc_segment_sum(e, dstT, N)

    h = x
    vn = jnp.zeros((G, D), dtype=jnp.float32)
    for l in range(L):
        hin = h if l == 0 else _hin(h, vn, batch2d, R)
        p = _sc_segment_sum(hin, dstT, N, srcT=srcT)
        h, g = _mlp(hin, p[0], p[1], es[0], es[1], batch2d,
                    W1[l], b1[l][None], W2[l], b2[l][None], R)
        vn = _vn_update(g, vn, Wv[l], bv[l][None])

    pp = _sc_segment_max(h, batchT)
    return _pool_combine(pp)


# speedup vs baseline: 7.7475x; 1.1374x over previous
import dataclasses
import functools

import jax
import jax.numpy as jnp
from jax import lax
from jax.experimental import pallas as pl
from jax.experimental.pallas import tpu as pltpu
from jax.experimental.pallas import tpu_sc as plsc

G = 256
NCORE = 2
NSUB = 16
NW = NCORE * NSUB
CHUNK = 80


def _sc_compiler_params():
    cp = pltpu.CompilerParams()
    if "needs_layout_passes" in pltpu.CompilerParams.__dataclass_fields__:
        cp = dataclasses.replace(cp, needs_layout_passes=False)
    return cp


def _edge_embed_body(ea_ref, we_ref, e_ref):
    e_ref[...] = jax.nn.relu(
        jnp.dot(ea_ref[...], we_ref[...], preferred_element_type=jnp.float32))


def _edge_embed(edge_attr, W_edge, block_e):
    E, DE = edge_attr.shape
    D = W_edge.shape[1]
    return pl.pallas_call(
        _edge_embed_body,
        grid=(E // block_e,),
        in_specs=[
            pl.BlockSpec((block_e, DE), lambda i: (i, 0)),
            pl.BlockSpec((DE, D), lambda i: (0, 0)),
        ],
        out_specs=pl.BlockSpec((block_e, D), lambda i: (i, 0)),
        out_shape=jax.ShapeDtypeStruct((E, D), jnp.float32),
    )(edge_attr, W_edge)


def _mlp_body(hin_ref, p0_ref, p1_ref, e0_ref, e1_ref, b_ref,
              w1_ref, b1_ref, w2_ref, b2_ref, hout_ref, g_ref, *, R):
    i = pl.program_id(0)
    hin = hin_ref[...]
    u = hin + p0_ref[...] + p1_ref[...] + e0_ref[...] + e1_ref[...]
    t = jax.nn.relu(
        jnp.dot(u, w1_ref[...], preferred_element_type=jnp.float32) + b1_ref[...])
    z = jnp.dot(t, w2_ref[...], preferred_element_type=jnp.float32) + b2_ref[...]
    h = hin + jax.nn.relu(z)
    hout_ref[...] = h
    P = (lax.broadcasted_iota(jnp.int32, (R, G), 1) == b_ref[...]).astype(jnp.float32)
    gg = lax.dot_general(P, h, (((0,), (0,)), ((), ())),
                         preferred_element_type=jnp.float32)

    @pl.when(i == 0)
    def _():
        g_ref[...] = gg

    @pl.when(i != 0)
    def _():
        g_ref[...] += gg


def _mlp(hin, p0, p1, e0, e1, batch2d, W1l, b1l, W2l, b2l, R):
    N, D = hin.shape
    D2 = W1l.shape[1]
    return pl.pallas_call(
        functools.partial(_mlp_body, R=R),
        grid=(N // R,),
        in_specs=[
            pl.BlockSpec((R, D), lambda i: (i, 0)),
            pl.BlockSpec((R, D), lambda i: (i, 0)),
            pl.BlockSpec((R, D), lambda i: (i, 0)),
            pl.BlockSpec((R, D), lambda i: (i, 0)),
            pl.BlockSpec((R, D), lambda i: (i, 0)),
            pl.BlockSpec((R, 1), lambda i: (i, 0)),
            pl.BlockSpec((D, D2), lambda i: (0, 0)),
            pl.BlockSpec((1, D2), lambda i: (0, 0)),
            pl.BlockSpec((D2, D), lambda i: (0, 0)),
            pl.BlockSpec((1, D), lambda i: (0, 0)),
        ],
        out_specs=[
            pl.BlockSpec((R, D), lambda i: (i, 0)),
            pl.BlockSpec((G, D), lambda i: (0, 0)),
        ],
        out_shape=[
            jax.ShapeDtypeStruct((N, D), jnp.float32),
            jax.ShapeDtypeStruct((G, D), jnp.float32),
        ],
    )(hin, p0, p1, e0, e1, batch2d, W1l, b1l, W2l, b2l)


def _vnhin_body(g_ref, vn_ref, wv_ref, bv_ref, h_ref, b_ref,
                vnout_ref, hin_ref, vns, *, R):
    i = pl.program_id(0)

    @pl.when(i == 0)
    def _():
        vn_next = vn_ref[...] + jax.nn.relu(
            jnp.dot(g_ref[...], wv_ref[...], preferred_element_type=jnp.float32)
            + bv_ref[...])
        vns[...] = vn_next
        vnout_ref[...] = vn_next

    P = (lax.broadcasted_iota(jnp.int32, (R, G), 1) == b_ref[...]).astype(jnp.float32)
    hin_ref[...] = h_ref[...] + jnp.dot(P, vns[...],
                                        preferred_element_type=jnp.float32)


def _vnhin(g, vn, Wvl, bvl, h, batch2d, R):
    N, D = h.shape
    return pl.pallas_call(
        functools.partial(_vnhin_body, R=R),
        grid=(N // R,),
        in_specs=[
            pl.BlockSpec((G, D), lambda i: (0, 0)),
            pl.BlockSpec((G, D), lambda i: (0, 0)),
            pl.BlockSpec((D, D), lambda i: (0, 0)),
            pl.BlockSpec((1, D), lambda i: (0, 0)),
            pl.BlockSpec((R, D), lambda i: (i, 0)),
            pl.BlockSpec((R, 1), lambda i: (i, 0)),
        ],
        out_specs=[
            pl.BlockSpec((G, D), lambda i: (0, 0)),
            pl.BlockSpec((R, D), lambda i: (i, 0)),
        ],
        out_shape=[
            jax.ShapeDtypeStruct((G, D), jnp.float32),
            jax.ShapeDtypeStruct((N, D), jnp.float32),
        ],
        scratch_shapes=[pltpu.VMEM((G, D), jnp.float32)],
    )(g, vn, Wvl, bvl, h, batch2d)


def _pool_combine_body(pp_ref, out_ref):
    m = jnp.max(pp_ref[...], axis=0)
    out_ref[...] = jnp.where(jnp.isfinite(m), m, 0.0)


def _pool_combine(pp):
    _, g, d = pp.shape
    return pl.pallas_call(
        _pool_combine_body,
        out_shape=jax.ShapeDtypeStruct((g, d), jnp.float32),
    )(pp)


def _sc_segment_sum(rows_hbm, dstT, N, srcT=None):
    D = rows_hbm.shape[1]
    NSC, SB, C = dstT.shape[1], dstT.shape[2], dstT.shape[3]
    NCH = NSC * SB
    gather = srcT is not None
    rpt = (N // NSUB) // 8 * 8
    tail = N - NSUB * rpt
    ZC, ZT = divmod(rpt, C)
    assert ZT % 8 == 0 and tail % 8 == 0 and tail <= C
    mesh = plsc.VectorSubcoreMesh(core_axis_name="c", subcore_axis_name="s")

    scratch = [
        pltpu.VMEM((SB, C), jnp.int32),
        pltpu.VMEM((C, D), jnp.float32),
        pltpu.VMEM((C, D), jnp.float32),
        pltpu.VMEM_SHARED((N, D), jnp.float32),
        pltpu.SemaphoreType.DMA,
        pltpu.SemaphoreType.DMA,
    ]
    if gather:
        scratch.insert(1, pltpu.VMEM((SB, C), jnp.int32))

    @functools.partial(
        pl.kernel,
        out_type=[jax.ShapeDtypeStruct((N, D), jnp.float32)] * NCORE,
        mesh=mesh,
        scratch_types=scratch,
        compiler_params=_sc_compiler_params(),
    )
    def k(*refs):
        if gather:
            rows_ref, src_hbm, dst_hbm, o0, o1, dst_v, src_v, r0, r1, acc, g0, g1 = refs
        else:
            rows_ref, dst_hbm, o0, o1, dst_v, r0, r1, acc, g0, g1 = refs
        c = lax.axis_index("c")
        s = lax.axis_index("s")
        wid = c * NSUB + s

        @pl.loop(0, C)
        def _(i):
            @pl.loop(0, D // 16)
            def _(j):
                r0[i, pl.ds(j * 16, 16)] = jnp.zeros((16,), jnp.float32)

        @pl.loop(0, ZC)
        def _(i):
            pltpu.sync_copy(r0, acc.at[pl.ds(s * rpt + i * C, C)])

        if ZT:
            pltpu.sync_copy(r0.at[pl.ds(0, ZT)],
                            acc.at[pl.ds(s * rpt + ZC * C, ZT)])

        if tail:
            @pl.when(s == NSUB - 1)
            def _():
                pltpu.sync_copy(r0.at[pl.ds(0, tail)],
                                acc.at[pl.ds(NSUB * rpt, tail)])

        plsc.subcore_barrier()

        assert SB % 2 == 1 and SB >= 3

        @pl.loop(0, NSC)
        def _(u):
            pltpu.sync_copy(dst_hbm.at[wid, u], dst_v)
            if gather:
                pltpu.sync_copy(src_hbm.at[wid, u], src_v)

            def load_copy(j, buf, sem):
                if gather:
                    return pltpu.make_async_copy(
                        rows_ref.at[src_v.at[j]], buf, sem)
                return pltpu.make_async_copy(
                    rows_ref.at[pl.ds(((wid * NSC + u) * SB + j) * C, C)],
                    buf, sem)

            def add_from(j, buf):
                pltpu.sync_copy(buf, acc.at[dst_v.at[j]], add=True)

            load_copy(0, r0, g0).start()
            load_copy(1, r1, g1).start()

            @pl.loop(0, SB - 1, step=2)
            def _(j):
                load_copy(j, r0, g0).wait()
                add_from(j, r0)
                load_copy(j + 2, r0, g0).start()
                load_copy(j + 1, r1, g1).wait()
                add_from(j + 1, r1)

                @pl.when(j + 3 < SB)
                def _():
                    load_copy(j + 3, r1, g1).start()

            load_copy(SB - 1, r0, g0).wait()
            add_from(SB - 1, r0)

        plsc.subcore_barrier()

        def dump(out_hbm):
            pltpu.sync_copy(acc.at[pl.ds(s * rpt, rpt)],
                            out_hbm.at[pl.ds(s * rpt, rpt)])
            if tail:
                @pl.when(s == NSUB - 1)
                def _():
                    pltpu.sync_copy(acc.at[pl.ds(NSUB * rpt, tail)],
                                    out_hbm.at[pl.ds(NSUB * rpt, tail)])

        @pl.when(c == 0)
        def _():
            dump(o0)

        @pl.when(c == 1)
        def _():
            dump(o1)

    if gather:
        return k(rows_hbm, srcT, dstT)
    return k(rows_hbm, dstT)


def _sc_segment_max(h, batchT):
    N, D = h.shape
    NCHUNKS, CH = batchT.shape
    STEPS = -(-NCHUNKS // NW)
    mesh = plsc.VectorSubcoreMesh(core_axis_name="c", subcore_axis_name="s")

    @functools.partial(
        pl.kernel,
        out_type=jax.ShapeDtypeStruct((NW, G, D), jnp.float32),
        mesh=mesh,
        scratch_types=[
            pltpu.VMEM((NCHUNKS, CH), jnp.int32),
            pltpu.VMEM((CH, D), jnp.float32),
            pltpu.VMEM((G, D), jnp.float32),
        ],
        compiler_params=_sc_compiler_params(),
    )
    def k(h_hbm, b_hbm, out_hbm, b_v, rows_v, acc):
        c = lax.axis_index("c")
        s = lax.axis_index("s")
        wid = c * NSUB + s

        @pl.loop(0, G)
        def _(i):
            @pl.loop(0, D // 16)
            def _(j):
                acc[i, pl.ds(j * 16, 16)] = jnp.full((16,), -jnp.inf, jnp.float32)

        pltpu.sync_copy(b_hbm, b_v)
        lane = lax.iota(jnp.int32, 16)

        @pl.loop(0, STEPS)
        def _(t):
            chunk = t * NW + wid

            @pl.when(chunk < NCHUNKS)
            def _():
                pltpu.sync_copy(h_hbm.at[pl.ds(chunk * CH, CH)], rows_v)
                bvec = b_v[chunk, pl.ds(0, CH)]
                for i in range(CH):
                    bi = jnp.max(jnp.where(lane == i, bvec, -1))
                    for j in range(D // 16):
                        sl = pl.ds(j * 16, 16)
                        acc[bi, sl] = jnp.maximum(acc[bi, sl], rows_v[i, sl])

        pltpu.sync_copy(acc, out_hbm.at[wid])

    return k(h, batchT)


def kernel(x, edge_attr, W_edge, W1, b1, W2, b2, Wv, bv, edge_index, batch):
    N, D = x.shape
    E = edge_attr.shape[0]
    L = W1.shape[0]
    R = 1000
    assert N % R == 0 and N % NSUB == 0 and N % 16 == 0
    NCH = E // (NW * CHUNK)
    assert NW * NCH * CHUNK == E
    SB = 25
    NSC = NCH // SB
    assert NSC * SB == NCH

    src = edge_index[0].astype(jnp.int32)
    dst = edge_index[1].astype(jnp.int32)
    srcT = src.reshape(NW, NSC, SB, CHUNK)
    dstT = dst.reshape(NW, NSC, SB, CHUNK)
    b32 = batch.astype(jnp.int32)
    batch2d = b32.reshape(N, 1)
    batchT = b32.reshape(N // 16, 16)

    p0, p1 = _sc_segment_sum(x, dstT, N, srcT=srcT)
    e = _edge_embed(edge_attr, W_edge, block_e=2000)
    es0, es1 = _sc_segment_sum(e, dstT, N)

    hin = x
    vn = jnp.zeros((G, D), dtype=jnp.float32)
    for l in range(L):
        if l > 0:
            p0, p1 = _sc_segment_sum(hin, dstT, N, srcT=srcT)
        h, g = _mlp(hin, p0, p1, es0, es1, batch2d,
                    W1[l], b1[l][None], W2[l], b2[l][None], R)
        if l + 1 < L:
            vn, hin = _vnhin(g, vn, Wv[l], bv[l][None], h, batch2d, R)

    pp = _sc_segment_max(h, batchT)
    return _pool_combine(pp)


# speedup vs baseline: 8.0887x; 1.0440x over previous
import dataclasses
import functools

import jax
import jax.numpy as jnp
from jax import lax
from jax.experimental import pallas as pl
from jax.experimental.pallas import tpu as pltpu
from jax.experimental.pallas import tpu_sc as plsc

G = 256
NCORE = 2
NSUB = 16
NW = NCORE * NSUB
CHUNK = 80


def _sc_compiler_params():
    cp = pltpu.CompilerParams()
    if "needs_layout_passes" in pltpu.CompilerParams.__dataclass_fields__:
        cp = dataclasses.replace(cp, needs_layout_passes=False)
    return cp


def _edge_embed_body(ea_ref, we_ref, e_ref, *, D):
    y = jax.nn.relu(
        jnp.dot(ea_ref[...].astype(jnp.bfloat16), we_ref[...],
                preferred_element_type=jnp.float32))
    for a in range(8):
        e_ref[:, a, :] = y[:, a * D:(a + 1) * D]


def _edge_embed(edge_attr, W_edge, block_e):
    E, DE = edge_attr.shape
    D = W_edge.shape[1]
    ea8 = edge_attr.reshape(E // 8, 8 * DE)
    wbig = jnp.kron(jnp.eye(8, dtype=jnp.float32), W_edge).astype(jnp.bfloat16)
    out = pl.pallas_call(
        functools.partial(_edge_embed_body, D=D),
        grid=(E // 8 // block_e,),
        in_specs=[
            pl.BlockSpec((block_e, 8 * DE), lambda i: (i, 0)),
            pl.BlockSpec((8 * DE, 8 * D), lambda i: (0, 0)),
        ],
        out_specs=pl.BlockSpec((block_e, 8, D), lambda i: (i, 0, 0)),
        out_shape=jax.ShapeDtypeStruct((E // 8, 8, D), jnp.float32),
    )(ea8, wbig)
    return out.reshape(E, D)


def _mlp_body(hin_ref, p0_ref, p1_ref, e0_ref, e1_ref, b_ref,
              w1_ref, b1_ref, w2_ref, b2_ref, hout_ref, g_ref, *, R):
    i = pl.program_id(0)
    hin = hin_ref[...]
    u = hin + p0_ref[...] + p1_ref[...] + e0_ref[...] + e1_ref[...]
    t = jax.nn.relu(
        jnp.dot(u, w1_ref[...], preferred_element_type=jnp.float32) + b1_ref[...])
    z = jnp.dot(t, w2_ref[...], preferred_element_type=jnp.float32) + b2_ref[...]
    h = hin + jax.nn.relu(z)
    hout_ref[...] = h
    P = (lax.broadcasted_iota(jnp.int32, (R, G), 1) == b_ref[...]).astype(jnp.float32)
    gg = lax.dot_general(P, h, (((0,), (0,)), ((), ())),
                         preferred_element_type=jnp.float32)

    @pl.when(i == 0)
    def _():
        g_ref[...] = gg

    @pl.when(i != 0)
    def _():
        g_ref[...] += gg


def _mlp(hin, p0, p1, e0, e1, batch2d, W1l, b1l, W2l, b2l, R):
    N, D = hin.shape
    D2 = W1l.shape[1]
    return pl.pallas_call(
        functools.partial(_mlp_body, R=R),
        grid=(N // R,),
        in_specs=[
            pl.BlockSpec((R, D), lambda i: (i, 0)),
            pl.BlockSpec((R, D), lambda i: (i, 0)),
            pl.BlockSpec((R, D), lambda i: (i, 0)),
            pl.BlockSpec((R, D), lambda i: (i, 0)),
            pl.BlockSpec((R, D), lambda i: (i, 0)),
            pl.BlockSpec((R, 1), lambda i: (i, 0)),
            pl.BlockSpec((D, D2), lambda i: (0, 0)),
            pl.BlockSpec((1, D2), lambda i: (0, 0)),
            pl.BlockSpec((D2, D), lambda i: (0, 0)),
            pl.BlockSpec((1, D), lambda i: (0, 0)),
        ],
        out_specs=[
            pl.BlockSpec((R, D), lambda i: (i, 0)),
            pl.BlockSpec((G, D), lambda i: (0, 0)),
        ],
        out_shape=[
            jax.ShapeDtypeStruct((N, D), jnp.float32),
            jax.ShapeDtypeStruct((G, D), jnp.float32),
        ],
    )(hin, p0, p1, e0, e1, batch2d, W1l, b1l, W2l, b2l)


def _vnhin_body(g_ref, vn_ref, wv_ref, bv_ref, h_ref, b_ref,
                vnout_ref, hin_ref, vns, *, R):
    i = pl.program_id(0)

    @pl.when(i == 0)
    def _():
        vn_next = vn_ref[...] + jax.nn.relu(
            jnp.dot(g_ref[...], wv_ref[...], preferred_element_type=jnp.float32)
            + bv_ref[...])
        vns[...] = vn_next
        vnout_ref[...] = vn_next

    P = (lax.broadcasted_iota(jnp.int32, (R, G), 1) == b_ref[...]).astype(jnp.float32)
    hin_ref[...] = h_ref[...] + jnp.dot(P, vns[...],
                                        preferred_element_type=jnp.float32)


def _vnhin(g, vn, Wvl, bvl, h, batch2d, R):
    N, D = h.shape
    return pl.pallas_call(
        functools.partial(_vnhin_body, R=R),
        grid=(N // R,),
        in_specs=[
            pl.BlockSpec((G, D), lambda i: (0, 0)),
            pl.BlockSpec((G, D), lambda i: (0, 0)),
            pl.BlockSpec((D, D), lambda i: (0, 0)),
            pl.BlockSpec((1, D), lambda i: (0, 0)),
            pl.BlockSpec((R, D), lambda i: (i, 0)),
            pl.BlockSpec((R, 1), lambda i: (i, 0)),
        ],
        out_specs=[
            pl.BlockSpec((G, D), lambda i: (0, 0)),
            pl.BlockSpec((R, D), lambda i: (i, 0)),
        ],
        out_shape=[
            jax.ShapeDtypeStruct((G, D), jnp.float32),
            jax.ShapeDtypeStruct((N, D), jnp.float32),
        ],
        scratch_shapes=[pltpu.VMEM((G, D), jnp.float32)],
    )(g, vn, Wvl, bvl, h, batch2d)


def _pool_combine_body(pp_ref, out_ref):
    m = jnp.max(pp_ref[...], axis=0)
    out_ref[...] = jnp.where(jnp.isfinite(m), m, 0.0)


def _pool_combine(pp):
    _, g, d = pp.shape
    return pl.pallas_call(
        _pool_combine_body,
        out_shape=jax.ShapeDtypeStruct((g, d), jnp.float32),
    )(pp)


def _sc_segment_sum(rows_hbm, dstT, N, srcT=None):
    D = rows_hbm.shape[1]
    NSC, SB, C = dstT.shape[1], dstT.shape[2], dstT.shape[3]
    NCH = NSC * SB
    gather = srcT is not None
    rpt = (N // NSUB) // 8 * 8
    tail = N - NSUB * rpt
    ZC, ZT = divmod(rpt, C)
    assert ZT % 8 == 0 and tail % 8 == 0 and tail <= C
    mesh = plsc.VectorSubcoreMesh(core_axis_name="c", subcore_axis_name="s")

    scratch = [
        pltpu.VMEM((SB, C), jnp.int32),
        pltpu.VMEM((C, D), jnp.float32),
        pltpu.VMEM((C, D), jnp.float32),
        pltpu.VMEM_SHARED((N, D), jnp.float32),
        pltpu.SemaphoreType.DMA,
        pltpu.SemaphoreType.DMA,
    ]
    if gather:
        scratch.insert(1, pltpu.VMEM((SB, C), jnp.int32))

    @functools.partial(
        pl.kernel,
        out_type=[jax.ShapeDtypeStruct((N, D), jnp.float32)] * NCORE,
        mesh=mesh,
        scratch_types=scratch,
        compiler_params=_sc_compiler_params(),
    )
    def k(*refs):
        if gather:
            rows_ref, src_hbm, dst_hbm, o0, o1, dst_v, src_v, r0, r1, acc, g0, g1 = refs
        else:
            rows_ref, dst_hbm, o0, o1, dst_v, r0, r1, acc, g0, g1 = refs
        c = lax.axis_index("c")
        s = lax.axis_index("s")
        wid = c * NSUB + s

        @pl.loop(0, C)
        def _(i):
            @pl.loop(0, D // 16)
            def _(j):
                r0[i, pl.ds(j * 16, 16)] = jnp.zeros((16,), jnp.float32)

        @pl.loop(0, ZC)
        def _(i):
            pltpu.sync_copy(r0, acc.at[pl.ds(s * rpt + i * C, C)])

        if ZT:
            pltpu.sync_copy(r0.at[pl.ds(0, ZT)],
                            acc.at[pl.ds(s * rpt + ZC * C, ZT)])

        if tail:
            @pl.when(s == NSUB - 1)
            def _():
                pltpu.sync_copy(r0.at[pl.ds(0, tail)],
                                acc.at[pl.ds(NSUB * rpt, tail)])

        plsc.subcore_barrier()

        assert SB % 2 == 1 and SB >= 3

        @pl.loop(0, NSC)
        def _(u):
            pltpu.sync_copy(dst_hbm.at[wid, u], dst_v)
            if gather:
                pltpu.sync_copy(src_hbm.at[wid, u], src_v)

            def load_copy(j, buf, sem):
                if gather:
                    return pltpu.make_async_copy(
                        rows_ref.at[src_v.at[j]], buf, sem)
                return pltpu.make_async_copy(
                    rows_ref.at[pl.ds(((wid * NSC + u) * SB + j) * C, C)],
                    buf, sem)

            def add_from(j, buf):
                pltpu.sync_copy(buf, acc.at[dst_v.at[j]], add=True)

            load_copy(0, r0, g0).start()
            load_copy(1, r1, g1).start()

            @pl.loop(0, SB - 1, step=2)
            def _(j):
                load_copy(j, r0, g0).wait()
                add_from(j, r0)
                load_copy(j + 2, r0, g0).start()
                load_copy(j + 1, r1, g1).wait()
                add_from(j + 1, r1)

                @pl.when(j + 3 < SB)
                def _():
                    load_copy(j + 3, r1, g1).start()

            load_copy(SB - 1, r0, g0).wait()
            add_from(SB - 1, r0)

        plsc.subcore_barrier()

        def dump(out_hbm):
            pltpu.sync_copy(acc.at[pl.ds(s * rpt, rpt)],
                            out_hbm.at[pl.ds(s * rpt, rpt)])
            if tail:
                @pl.when(s == NSUB - 1)
                def _():
                    pltpu.sync_copy(acc.at[pl.ds(NSUB * rpt, tail)],
                                    out_hbm.at[pl.ds(NSUB * rpt, tail)])

        @pl.when(c == 0)
        def _():
            dump(o0)

        @pl.when(c == 1)
        def _():
            dump(o1)

    if gather:
        return k(rows_hbm, srcT, dstT)
    return k(rows_hbm, dstT)


def _sc_segment_max(h, batchT):
    N, D = h.shape
    NCHUNKS, CH = batchT.shape
    STEPS = -(-NCHUNKS // NW)
    mesh = plsc.VectorSubcoreMesh(core_axis_name="c", subcore_axis_name="s")

    @functools.partial(
        pl.kernel,
        out_type=jax.ShapeDtypeStruct((NW, G, D), jnp.float32),
        mesh=mesh,
        scratch_types=[
            pltpu.VMEM((NCHUNKS, CH), jnp.int32),
            pltpu.VMEM((CH, D), jnp.float32),
            pltpu.VMEM((G, D), jnp.float32),
        ],
        compiler_params=_sc_compiler_params(),
    )
    def k(h_hbm, b_hbm, out_hbm, b_v, rows_v, acc):
        c = lax.axis_index("c")
        s = lax.axis_index("s")
        wid = c * NSUB + s

        @pl.loop(0, G)
        def _(i):
            @pl.loop(0, D // 16)
            def _(j):
                acc[i, pl.ds(j * 16, 16)] = jnp.full((16,), -jnp.inf, jnp.float32)

        pltpu.sync_copy(b_hbm, b_v)
        lane = lax.iota(jnp.int32, 16)

        @pl.loop(0, STEPS)
        def _(t):
            chunk = t * NW + wid

            @pl.when(chunk < NCHUNKS)
            def _():
                pltpu.sync_copy(h_hbm.at[pl.ds(chunk * CH, CH)], rows_v)
                bvec = b_v[chunk, pl.ds(0, CH)]
                for i in range(CH):
                    bi = jnp.max(jnp.where(lane == i, bvec, -1))
                    for j in range(D // 16):
                        sl = pl.ds(j * 16, 16)
                        acc[bi, sl] = jnp.maximum(acc[bi, sl], rows_v[i, sl])

        pltpu.sync_copy(acc, out_hbm.at[wid])

    return k(h, batchT)


def kernel(x, edge_attr, W_edge, W1, b1, W2, b2, Wv, bv, edge_index, batch):
    N, D = x.shape
    E = edge_attr.shape[0]
    L = W1.shape[0]
    R = 1000
    assert N % R == 0 and N % NSUB == 0 and N % 16 == 0
    NCH = E // (NW * CHUNK)
    assert NW * NCH * CHUNK == E
    SB = 25
    NSC = NCH // SB
    assert NSC * SB == NCH

    src = edge_index[0].astype(jnp.int32)
    dst = edge_index[1].astype(jnp.int32)
    srcT = src.reshape(NW, NSC, SB, CHUNK)
    dstT = dst.reshape(NW, NSC, SB, CHUNK)
    b32 = batch.astype(jnp.int32)
    batch2d = b32.reshape(N, 1)
    batchT = b32.reshape(N // 16, 16)

    p0, p1 = _sc_segment_sum(x, dstT, N, srcT=srcT)
    e = _edge_embed(edge_attr, W_edge, block_e=1000)
    es0, es1 = _sc_segment_sum(e, dstT, N)

    hin = x
    vn = jnp.zeros((G, D), dtype=jnp.float32)
    for l in range(L):
        if l > 0:
            p0, p1 = _sc_segment_sum(hin, dstT, N, srcT=srcT)
        h, g = _mlp(hin, p0, p1, es0, es1, batch2d,
                    W1[l], b1[l][None], W2[l], b2[l][None], R)
        if l + 1 < L:
            vn, hin = _vnhin(g, vn, Wv[l], bv[l][None], h, batch2d, R)

    pp = _sc_segment_max(h, batchT)
    return _pool_combine(pp)
